# baseline (speedup 1.0000x reference)
import functools
import math

import jax
import jax.numpy as jnp
from jax import lax
from jax.experimental import pallas as pl
from jax.experimental.pallas import tpu as pltpu
from jax.experimental.pallas import tpu_sc as plsc

H = 128
LEVELS = ('symbol', 'term', 'literal', 'clause', 'proof')
INTER_LIST = (('s2t', 'symbol', 'term'), ('t2l', 'term', 'literal'),
              ('l2c', 'literal', 'clause'), ('c2p', 'clause', 'proof'))
NUM_LEVELS = 5
HEADS = 4
DH = H // HEADS
BN = 256
C = 128
W = 32


def _rup(a, b):
    return (a + b - 1) // b * b


@functools.lru_cache(None)
def _segsum_call(n_pad, e_pad):
    n_iter = e_pad // (W * C)
    per_w = e_pad // W
    rpt = n_pad // 16
    mesh = plsc.VectorSubcoreMesh(core_axis_name="c", subcore_axis_name="s")

    def body(tbl, src, dst, out, src_v, dst_v, rows_v, zbuf, acc, sem):
        cid = lax.axis_index("c")
        sid = lax.axis_index("s")
        wid = sid * 2 + cid
        for i in range(8):
            for j in range(H // 16):
                zbuf[i, pl.ds(j * 16, 16)] = jnp.zeros((16,), jnp.float32)
        row0 = sid * rpt

        def zloop(r, carry):
            pltpu.sync_copy(zbuf, acc.at[pl.ds(row0 + r * 8, 8)])
            return carry
        lax.fori_loop(0, rpt // 8, zloop, 0)
        plsc.subcore_barrier()

        base0 = wid * per_w

        def eloop(i, carry):
            b = base0 + i * C
            pltpu.sync_copy(src.at[pl.ds(b, C)], src_v)
            pltpu.sync_copy(dst.at[pl.ds(b, C)], dst_v)
            pltpu.async_copy(tbl.at[src_v], rows_v, sem).wait()
            pltpu.sync_copy(rows_v, acc.at[dst_v], add=True)
            return carry
        lax.fori_loop(0, n_iter, eloop, 0)
        plsc.subcore_barrier()
        pltpu.sync_copy(acc.at[pl.ds(row0, rpt)],
                        out.at[cid, pl.ds(row0, rpt)])

    return pl.kernel(
        body, mesh=mesh,
        out_type=jax.ShapeDtypeStruct((2, n_pad, H), jnp.float32),
        scratch_types=[
            pltpu.VMEM((C,), jnp.int32),
            pltpu.VMEM((C,), jnp.int32),
            pltpu.VMEM((C, H), jnp.float32),
            pltpu.VMEM((8, H), jnp.float32),
            pltpu.VMEM_SHARED((n_pad, H), jnp.float32),
            pltpu.SemaphoreType.DMA,
        ])


def _count_partials(n_pad, e_pad, dst):
    ones_tbl = jnp.ones((8, H), jnp.float32)
    zsrc = jnp.zeros(dst.shape, jnp.int32)
    full = _segsum_call(n_pad, e_pad)(ones_tbl, zsrc, dst)
    return full[:, :, :16]


def _dense_body(x_ref, p_ref, c_ref, ws_ref, wn_ref, b_ref, o_ref):
    cnt = jnp.maximum(c_ref[0, :, :1] + c_ref[1, :, :1], 1.0)
    m = (p_ref[0] + p_ref[1]) / cnt
    o_ref[...] = jnp.maximum(
        jnp.dot(x_ref[...], ws_ref[...], preferred_element_type=jnp.float32)
        + jnp.dot(m, wn_ref[...], preferred_element_type=jnp.float32)
        + b_ref[...], 0.0)


@functools.lru_cache(None)
def _dense_call(n_pad):
    g = n_pad // BN
    return pl.pallas_call(
        _dense_body,
        grid=(g,),
        in_specs=[
            pl.BlockSpec((BN, H), lambda i: (i, 0)),
            pl.BlockSpec((2, BN, H), lambda i: (0, i, 0)),
            pl.BlockSpec((2, BN, 16), lambda i: (0, i, 0)),
            pl.BlockSpec((H, H), lambda i: (0, 0)),
            pl.BlockSpec((H, H), lambda i: (0, 0)),
            pl.BlockSpec((1, H), lambda i: (0, 0)),
        ],
        out_specs=pl.BlockSpec((BN, H), lambda i: (i, 0)),
        out_shape=jax.ShapeDtypeStruct((n_pad, H), jnp.float32),
    )


def _inter_body(h_ref, p_ref, c_ref, w_ref, o_ref):
    cnt = jnp.maximum(c_ref[0, :, :1] + c_ref[1, :, :1], 1.0)
    agg = (p_ref[0] + p_ref[1]) / cnt
    o_ref[...] = jnp.maximum(
        h_ref[...]
        + jnp.dot(agg, w_ref[...], preferred_element_type=jnp.float32), 0.0)


@functools.lru_cache(None)
def _inter_call(n_pad):
    g = n_pad // BN
    return pl.pallas_call(
        _inter_body,
        grid=(g,),
        in_specs=[
            pl.BlockSpec((BN, H), lambda i: (i, 0)),
            pl.BlockSpec((2, BN, H), lambda i: (0, i, 0)),
            pl.BlockSpec((2, BN, 16), lambda i: (0, i, 0)),
            pl.BlockSpec((H, H), lambda i: (0, 0)),
        ],
        out_specs=pl.BlockSpec((BN, H), lambda i: (i, 0)),
        out_shape=jax.ShapeDtypeStruct((n_pad, H), jnp.float32),
    )


@functools.lru_cache(None)
def _mean_call(n_pad, n_real):
    g = n_pad // BN

    def body(x_ref, o_ref):
        i = pl.program_id(0)

        @pl.when(i == 0)
        def _():
            o_ref[...] = jnp.zeros_like(o_ref)

        rows = i * BN + lax.broadcasted_iota(jnp.int32, (BN, 1), 0)
        xm = jnp.where(rows < n_real, x_ref[...], 0.0)
        o_ref[...] += jnp.sum(xm, axis=0, keepdims=True) / n_real

    return pl.pallas_call(
        body,
        grid=(g,),
        in_specs=[pl.BlockSpec((BN, H), lambda i: (i, 0))],
        out_specs=pl.BlockSpec((1, H), lambda i: (0, 0)),
        out_shape=jax.ShapeDtypeStruct((1, H), jnp.float32),
    )


def _attn_body(h_ref, s_ref, wq_ref, wk_ref, wv_ref, wo_ref, o_ref):
    f32 = jnp.float32
    k5 = jnp.dot(s_ref[...], wk_ref[...], preferred_element_type=f32)
    v5 = jnp.dot(s_ref[...], wv_ref[...], preferred_element_type=f32)
    q = jnp.dot(h_ref[...], wq_ref[...], preferred_element_type=f32)
    hr = lax.broadcasted_iota(jnp.int32, (H, H), 0) // DH
    hc = lax.broadcasted_iota(jnp.int32, (H, H), 1) // DH
    bsum = (hr == hc).astype(f32)
    scale = 1.0 / math.sqrt(DH)
    logits = [jnp.dot(q * k5[l:l + 1, :], bsum, preferred_element_type=f32)
              * scale for l in range(NUM_LEVELS)]
    mx = logits[0]
    for l in range(1, NUM_LEVELS):
        mx = jnp.maximum(mx, logits[l])
    es = [jnp.exp(sl - mx) for sl in logits]
    den = es[0] + es[1] + es[2] + es[3] + es[4]
    ctx = sum(es[l] * v5[l:l + 1, :] for l in range(NUM_LEVELS)) / den
    o_ref[...] = h_ref[...] + jnp.dot(ctx, wo_ref[...],
                                      preferred_element_type=f32)


@functools.lru_cache(None)
def _attn_call(n_pad):
    g = n_pad // BN
    return pl.pallas_call(
        _attn_body,
        grid=(g,),
        in_specs=[
            pl.BlockSpec((BN, H), lambda i: (i, 0)),
            pl.BlockSpec((8, H), lambda i: (0, 0)),
            pl.BlockSpec((H, H), lambda i: (0, 0)),
            pl.BlockSpec((H, H), lambda i: (0, 0)),
            pl.BlockSpec((H, H), lambda i: (0, 0)),
            pl.BlockSpec((H, H), lambda i: (0, 0)),
        ],
        out_specs=pl.BlockSpec((BN, H), lambda i: (i, 0)),
        out_shape=jax.ShapeDtypeStruct((n_pad, H), jnp.float32),
    )


def _outp_body(h_ref, w_ref, b_ref, o_ref):
    o_ref[...] = jnp.dot(h_ref[...], w_ref[...],
                         preferred_element_type=jnp.float32) + b_ref[...]


@functools.lru_cache(None)
def _outp_call(n_pad, e_out):
    g = n_pad // BN
    return pl.pallas_call(
        _outp_body,
        grid=(g,),
        in_specs=[
            pl.BlockSpec((BN, H), lambda i: (i, 0)),
            pl.BlockSpec((H, e_out), lambda i: (0, 0)),
            pl.BlockSpec((1, e_out), lambda i: (0, 0)),
        ],
        out_specs=pl.BlockSpec((BN, e_out), lambda i: (i, 0)),
        out_shape=jax.ShapeDtypeStruct((n_pad, e_out), jnp.float32),
    )


def kernel(x_symbol, edge_symbol, W_self_symbol, W_nbr_symbol, b_symbol,
           x_term, edge_term, W_self_term, W_nbr_term, b_term,
           x_literal, edge_literal, W_self_literal, W_nbr_literal, b_literal,
           x_clause, edge_clause, W_self_clause, W_nbr_clause, b_clause,
           x_proof, edge_proof, W_self_proof, W_nbr_proof, b_proof,
           s2t_src, s2t_dst, W_inter_s2t,
           t2l_src, t2l_dst, W_inter_t2l,
           l2c_src, l2c_dst, W_inter_l2c,
           c2p_src, c2p_dst, W_inter_c2p,
           Wq, Wk, Wv, Wo, W_out, b_out):
    d = dict(locals())

    h, n_pad, n_real, ei, cnts = {}, {}, {}, {}, {}
    for lvl in LEVELS:
        n = d['x_' + lvl].shape[0]
        npd = _rup(n + 1, BN)
        n_real[lvl] = n
        n_pad[lvl] = npd
        h[lvl] = jnp.pad(d['x_' + lvl], ((0, npd - n), (0, 0)))
        e = d['edge_' + lvl].shape[1]
        ep = _rup(e, W * C)
        src = jnp.pad(d['edge_' + lvl][0], (0, ep - e))
        dst = jnp.pad(d['edge_' + lvl][1], (0, ep - e), constant_values=n)
        ei[lvl] = (src, dst, ep)
        cnts[lvl] = _count_partials(npd, ep, dst)
    for name, lo, hi in INTER_LIST:
        e = d[name + '_src'].shape[0]
        ep = _rup(e, W * C)
        src = jnp.pad(d[name + '_src'], (0, ep - e))
        dst = jnp.pad(d[name + '_dst'], (0, ep - e),
                      constant_values=n_real[hi])
        ei[name] = (src, dst, ep)
        cnts[name] = _count_partials(n_pad[hi], ep, dst)

    for _rnd in range(2):
        for lvl in LEVELS:
            src, dst, ep = ei[lvl]
            for _l in range(3):
                part = _segsum_call(n_pad[lvl], ep)(h[lvl], src, dst)
                h[lvl] = _dense_call(n_pad[lvl])(
                    h[lvl], part, cnts[lvl], d['W_self_' + lvl],
                    d['W_nbr_' + lvl], d['b_' + lvl].reshape(1, H))
        for name, lo, hi in INTER_LIST:
            src, dst, ep = ei[name]
            part = _segsum_call(n_pad[hi], ep)(h[lo], src, dst)
            h[hi] = _inter_call(n_pad[hi])(
                h[hi], part, cnts[name], d['W_inter_' + name])
        summ = jnp.concatenate(
            [_mean_call(n_pad[lvl], n_real[lvl])(h[lvl]) for lvl in LEVELS]
            + [jnp.zeros((8 - NUM_LEVELS, H), jnp.float32)], axis=0)
        for lvl in LEVELS:
            h[lvl] = _attn_call(n_pad[lvl])(
                h[lvl], summ, d['Wq'], d['Wk'], d['Wv'], d['Wo'])

    e_out = W_out.shape[1]
    out = _outp_call(n_pad['clause'], e_out)(
        h['clause'], W_out, b_out.reshape(1, e_out))
    return out[:n_real['clause']]
